# Initial kernel scaffold; baseline (speedup 1.0000x reference)
import functools

import jax
import jax.numpy as jnp
from jax import lax
from jax.experimental import pallas as pl
from jax.experimental.pallas import tpu as pltpu
from jax.experimental.pallas import tpu_sc as plsc

_D = 32
_NC = 2
_NS = 16
_NW = _NC * _NS
_CHUNK = 128


@functools.lru_cache(maxsize=None)
def _make_gather(B: int):
    bpw = B // _NW
    nchunk = bpw // _CHUNK
    mesh = plsc.VectorSubcoreMesh(core_axis_name="c", subcore_axis_name="s")

    @functools.partial(
        pl.kernel,
        mesh=mesh,
        out_type=jax.ShapeDtypeStruct((B, _D), jnp.float32),
        scratch_types=[
            pltpu.VMEM((nchunk, _CHUNK), jnp.int32),
            pltpu.VMEM((2, _CHUNK, _D), jnp.float32),
            pltpu.SemaphoreType.DMA,
            pltpu.SemaphoreType.DMA,
        ],
    )
    def gather(idx_hbm, table_hbm, out_hbm, idx_v, rows_v, gsem, ssem):
        wid = lax.axis_index("s") * _NC + lax.axis_index("c")
        base = wid * bpw
        pltpu.sync_copy(idx_hbm.at[wid], idx_v)

        def body(j, carry):
            pltpu.async_copy(table_hbm.at[idx_v.at[j]], rows_v.at[0], gsem).wait()
            pltpu.sync_copy(
                rows_v.at[0], out_hbm.at[pl.ds(base + j * _CHUNK, _CHUNK)]
            )
            return carry

        lax.fori_loop(0, nchunk, body, 0)

    return gather


def kernel(indices, W):
    rows, cols = indices.shape
    B = rows * cols
    idx3 = indices.reshape(_NW, (B // _NW) // _CHUNK, _CHUNK)
    out = _make_gather(B)(idx3, W)
    return out.reshape(rows, cols, _D)


# speedup vs baseline: 1.3067x; 1.3067x over previous
import functools

import jax
import jax.numpy as jnp
from jax import lax
from jax.experimental import pallas as pl
from jax.experimental.pallas import tpu as pltpu
from jax.experimental.pallas import tpu_sc as plsc

_D = 32
_NC = 2
_NS = 16
_NW = _NC * _NS
_CHUNK = 128


@functools.lru_cache(maxsize=None)
def _make_gather(B: int):
    bpw = B // _NW
    nchunk = bpw // _CHUNK
    mesh = plsc.VectorSubcoreMesh(core_axis_name="c", subcore_axis_name="s")

    @functools.partial(
        pl.kernel,
        mesh=mesh,
        out_type=jax.ShapeDtypeStruct((B, _D), jnp.float32),
        scratch_types=[
            pltpu.VMEM((nchunk, _CHUNK), jnp.int32),
            pltpu.VMEM((2, _CHUNK, _D), jnp.float32),
            pltpu.SemaphoreType.DMA,
            pltpu.SemaphoreType.DMA,
        ],
        compiler_params=pltpu.CompilerParams(use_tc_tiling_on_sc=False),
    )
    def gather(idx_hbm, table_hbm, out_hbm, idx_v, rows_v, gsem, ssem):
        wid = lax.axis_index("s") * _NC + lax.axis_index("c")
        base = wid * bpw
        pltpu.sync_copy(idx_hbm.at[wid], idx_v)

        def body(j, carry):
            pltpu.async_copy(table_hbm.at[idx_v.at[j]], rows_v.at[0], gsem).wait()
            pltpu.sync_copy(
                rows_v.at[0], out_hbm.at[pl.ds(base + j * _CHUNK, _CHUNK)]
            )
            return carry

        lax.fori_loop(0, nchunk, body, 0)

    return gather


def kernel(indices, W):
    rows, cols = indices.shape
    B = rows * cols
    idx3 = indices.reshape(_NW, (B // _NW) // _CHUNK, _CHUNK)
    out = _make_gather(B)(idx3, W)
    return out.reshape(rows, cols, _D)


# speedup vs baseline: 1.5025x; 1.1499x over previous
import functools

import jax
import jax.numpy as jnp
from jax import lax
from jax.experimental import pallas as pl
from jax.experimental.pallas import tpu as pltpu
from jax.experimental.pallas import tpu_sc as plsc

_D = 32
_NC = 2
_NS = 16
_NW = _NC * _NS
_CHUNK = 128


_SGRP = 4
_GROUP = _SGRP * _CHUNK
_NB = 5


@functools.lru_cache(maxsize=None)
def _make_gather(B: int):
    bpw = B // _NW
    nchunk = bpw // _CHUNK
    ngrp = nchunk // _SGRP
    mesh = plsc.VectorSubcoreMesh(core_axis_name="c", subcore_axis_name="s")

    @functools.partial(
        pl.kernel,
        mesh=mesh,
        out_type=jax.ShapeDtypeStruct((B, _D), jnp.float32),
        scratch_types=[
            pltpu.VMEM((nchunk, _CHUNK), jnp.int32),
            pltpu.VMEM((_NB, _GROUP, _D), jnp.float32),
            pltpu.SemaphoreType.DMA((_NB, _SGRP)),
        ],
        compiler_params=pltpu.CompilerParams(use_tc_tiling_on_sc=False),
    )
    def gather(idx_hbm, table_hbm, out_hbm, idx_v, rows_v, gsem):
        wid = lax.axis_index("s") * _NC + lax.axis_index("c")
        base = wid * bpw
        pltpu.sync_copy(idx_hbm.at[wid], idx_v)

        def gather_group(g, b):
            for s in range(_SGRP):
                pltpu.make_async_copy(
                    table_hbm.at[idx_v.at[g * _SGRP + s]],
                    rows_v.at[b, pl.ds(s * _CHUNK, _CHUNK)],
                    gsem.at[b, s],
                ).start()

        def wait_group(g, b):
            for s in range(_SGRP):
                pltpu.make_async_copy(
                    table_hbm.at[idx_v.at[g * _SGRP + s]],
                    rows_v.at[b, pl.ds(s * _CHUNK, _CHUNK)],
                    gsem.at[b, s],
                ).wait()

        for b in range(_NB):
            gather_group(b, b)

        def outer(go, carry):
            for b in range(_NB):
                g = go * _NB + b
                wait_group(g, b)
                pltpu.sync_copy(
                    rows_v.at[b], out_hbm.at[pl.ds(base + g * _GROUP, _GROUP)]
                )

                @pl.when(g + _NB < ngrp)
                def _():
                    gather_group(g + _NB, b)

            return carry

        lax.fori_loop(0, ngrp // _NB, outer, 0)

    return gather


def kernel(indices, W):
    rows, cols = indices.shape
    B = rows * cols
    idx3 = indices.reshape(_NW, (B // _NW) // _CHUNK, _CHUNK)
    out = _make_gather(B)(idx3, W)
    return out.reshape(rows, cols, _D)
